# baseline (speedup 1.0000x reference)
import functools

import jax
import jax.numpy as jnp
from jax import lax
from jax.experimental import pallas as pl
from jax.experimental.pallas import tpu as pltpu
from jax.experimental.pallas import tpu_sc as plsc

K = 8192
D = 256
NTOK = 16 * 576

TT = 512
KT = 1024

NW = 32
BPW = NTOK // NW
CH = 96
NCH = BPW // CH


def _encode_body(zt_ref, zn_ref, cn_ref, cb_ref, idx_ref):
    zb = zt_ref[...]
    zn = zn_ref[...]
    bv = jnp.full((TT, 1), jnp.inf, jnp.float32)
    bi = jnp.zeros((TT, 1), jnp.int32)
    for k in range(K // KT):
        c = cb_ref[pl.ds(k * KT, KT), :]
        s = lax.dot_general(zb, c, (((1,), (1,)), ((), ())),
                            preferred_element_type=jnp.float32)
        cn = cn_ref[:, pl.ds(k * KT, KT)]
        score = (zn - 2.0 * s) + cn
        m = jnp.min(score, axis=1, keepdims=True)
        ids = lax.broadcasted_iota(jnp.int32, (TT, KT), 1) + (k * KT)
        am = jnp.min(jnp.where(score == m, ids, jnp.int32(2 ** 30)),
                     axis=1, keepdims=True)
        upd = m < bv
        bv = jnp.where(upd, m, bv)
        bi = jnp.where(upd, am, bi)
    idx_ref[...] = bi


def _encode(zt, zn, cn, codebook):
    return pl.pallas_call(
        _encode_body,
        grid=(NTOK // TT,),
        in_specs=[
            pl.BlockSpec((TT, D), lambda i: (i, 0)),
            pl.BlockSpec((TT, 1), lambda i: (i, 0)),
            pl.BlockSpec((1, K), lambda i: (0, 0)),
            pl.BlockSpec((K, D), lambda i: (0, 0)),
        ],
        out_specs=pl.BlockSpec((TT, 1), lambda i: (i, 0)),
        out_shape=jax.ShapeDtypeStruct((NTOK, 1), jnp.int32),
    )(zt, zn, cn, codebook)


_SC_MESH = plsc.VectorSubcoreMesh(core_axis_name="c", subcore_axis_name="s")


@functools.partial(
    pl.kernel,
    mesh=_SC_MESH,
    out_type=jax.ShapeDtypeStruct((NTOK, D), jnp.float32),
    scratch_types=[
        pltpu.VMEM((NCH, CH), jnp.int32),
        pltpu.VMEM((BPW, D), jnp.float32),
        pltpu.SemaphoreType.DMA,
    ],
)
def _decode_sc(cb_hbm, idx_hbm, out_hbm, idx_v, rows_v, sem):
    wid = lax.axis_index("s") * 2 + lax.axis_index("c")
    pltpu.sync_copy(idx_hbm.at[wid], idx_v)
    copies = [
        pltpu.async_copy(cb_hbm.at[idx_v.at[j]],
                         rows_v.at[pl.ds(j * CH, CH)], sem)
        for j in range(NCH)
    ]
    for cp in copies:
        cp.wait()
    pltpu.sync_copy(rows_v, out_hbm.at[pl.ds(wid * BPW, BPW)])


def kernel(z, codebook):
    B, _, T = z.shape
    zt = jnp.transpose(z, (0, 2, 1)).reshape(-1, D)
    zn = jnp.sum(zt * zt, axis=1, keepdims=True)
    cn = jnp.sum(codebook * codebook, axis=1)[None, :]
    idx = _encode(zt, zn, cn, codebook)
    idx3 = idx.reshape(NW, NCH, CH)
    q = _decode_sc(codebook, idx3)
    return jnp.transpose(q.reshape(B, T, D), (0, 2, 1))


# speedup vs baseline: 1.2031x; 1.2031x over previous
import functools

import jax
import jax.numpy as jnp
from jax import lax
from jax.experimental import pallas as pl
from jax.experimental.pallas import tpu as pltpu
from jax.experimental.pallas import tpu_sc as plsc

K = 8192
D = 256
NTOK = 16 * 576

TT = 512
KT = 1024

NW = 32
BPW = NTOK // NW
CH = 96
NCH = BPW // CH


def _encode_body(zt_ref, zn_ref, cn_ref, cbm2_ref, idx_ref):
    zb = zt_ref[...]
    zn = zn_ref[...]
    posf = lax.broadcasted_iota(jnp.int32, (TT, KT), 1).astype(jnp.float32)
    bv = jnp.full((TT, 1), jnp.inf, jnp.float32)
    bif = jnp.zeros((TT, 1), jnp.float32)
    for k in range(K // KT):
        c2 = cbm2_ref[pl.ds(k * KT, KT), :]
        s2 = lax.dot_general(zb, c2, (((1,), (1,)), ((), ())),
                             preferred_element_type=jnp.float32)
        cn = cn_ref[:, pl.ds(k * KT, KT)]
        score = (zn + s2) + cn
        m = jnp.min(score, axis=1, keepdims=True)
        am = jnp.min(jnp.where(score == m, posf, jnp.inf),
                     axis=1, keepdims=True)
        upd = m < bv
        bv = jnp.where(upd, m, bv)
        bif = jnp.where(upd, am + jnp.float32(k * KT), bif)
    idx_ref[...] = bif.astype(jnp.int32)


def _encode(zt, zn, cn, codebook):
    return pl.pallas_call(
        _encode_body,
        grid=(NTOK // TT,),
        in_specs=[
            pl.BlockSpec((TT, D), lambda i: (i, 0)),
            pl.BlockSpec((TT, 1), lambda i: (i, 0)),
            pl.BlockSpec((1, K), lambda i: (0, 0)),
            pl.BlockSpec((K, D), lambda i: (0, 0)),
        ],
        out_specs=pl.BlockSpec((TT, 1), lambda i: (i, 0)),
        out_shape=jax.ShapeDtypeStruct((NTOK, 1), jnp.int32),
    )(zt, zn, cn, codebook)


_SC_MESH = plsc.VectorSubcoreMesh(core_axis_name="c", subcore_axis_name="s")


@functools.partial(
    pl.kernel,
    mesh=_SC_MESH,
    out_type=jax.ShapeDtypeStruct((NTOK, D), jnp.float32),
    scratch_types=[
        pltpu.VMEM((NCH, CH), jnp.int32),
        pltpu.VMEM((BPW, D), jnp.float32),
        pltpu.SemaphoreType.DMA,
    ],
)
def _decode_sc(cb_hbm, idx_hbm, out_hbm, idx_v, rows_v, sem):
    wid = lax.axis_index("s") * 2 + lax.axis_index("c")
    pltpu.sync_copy(idx_hbm.at[wid], idx_v)
    copies = [
        pltpu.async_copy(cb_hbm.at[idx_v.at[j]],
                         rows_v.at[pl.ds(j * CH, CH)], sem)
        for j in range(NCH)
    ]
    for cp in copies:
        cp.wait()
    pltpu.sync_copy(rows_v, out_hbm.at[pl.ds(wid * BPW, BPW)])


def kernel(z, codebook):
    B, _, T = z.shape
    zt = jnp.transpose(z, (0, 2, 1)).reshape(-1, D)
    zn = jnp.sum(zt * zt, axis=1, keepdims=True)
    cn = jnp.sum(codebook * codebook, axis=1)[None, :]
    idx = _encode(zt, zn, cn, -2.0 * codebook)
    idx3 = idx.reshape(NW, NCH, CH)
    q = _decode_sc(codebook, idx3)
    return jnp.transpose(q.reshape(B, T, D), (0, 2, 1))
